# baseline (speedup 1.0000x reference)
import functools

import jax
import jax.numpy as jnp
from jax import lax
from jax.experimental import pallas as pl
from jax.experimental.pallas import tpu as pltpu
from jax.experimental.pallas import tpu_sc as plsc

N_EXPERTS = 16
TOP2 = 2
LANES = 16

ROW_BLOCK = 512
NBUF = 6


def _router_probs_body(x_hbm, w_ref, p_ref, pt_ref, x_buf, sems):
    i = pl.program_id(0)
    steps = pl.num_programs(0)

    def copy_block(blk, slot):
        return pltpu.make_async_copy(
            x_hbm.at[pl.ds(blk * ROW_BLOCK, ROW_BLOCK), :],
            x_buf.at[slot],
            sems.at[slot],
        )

    @pl.when(i == 0)
    def _():
        for b in range(NBUF - 1):
            copy_block(b, b).start()

    @pl.when(i + NBUF - 1 < steps)
    def _():
        copy_block(i + NBUF - 1, lax.rem(i + NBUF - 1, NBUF)).start()

    slot = lax.rem(i, NBUF)
    copy_block(i, slot).wait()
    x = x_buf[slot]
    w = w_ref[...]
    logits = lax.dot_general(x, w, (((1,), (1,)), ((), ())),
                             preferred_element_type=jnp.float32)
    m = jnp.max(logits, axis=1, keepdims=True)
    e = jnp.exp(logits - m)
    p = e / jnp.sum(e, axis=1, keepdims=True)
    p_ref[...] = p
    pt_ref[...] = p.T[None]


def _router_probs(x, w_router, row_block):
    tokens, d_model = x.shape
    steps = tokens // row_block
    return pl.pallas_call(
        _router_probs_body,
        grid=(steps,),
        in_specs=[
            pl.BlockSpec(memory_space=pl.ANY),
            pl.BlockSpec((N_EXPERTS, d_model), lambda i: (0, 0)),
        ],
        out_specs=[
            pl.BlockSpec((row_block, N_EXPERTS), lambda i: (i, 0)),
            pl.BlockSpec((1, N_EXPERTS, row_block), lambda i: (i, 0, 0)),
        ],
        out_shape=[
            jax.ShapeDtypeStruct((tokens, N_EXPERTS), jnp.float32),
            jax.ShapeDtypeStruct((steps, N_EXPERTS, row_block), jnp.float32),
        ],
        scratch_shapes=[
            pltpu.VMEM((NBUF, row_block, d_model), jnp.float32),
            pltpu.SemaphoreType.DMA((NBUF,)),
        ],
        compiler_params=pltpu.CompilerParams(vmem_limit_bytes=128 * 1024 * 1024),
    )(x, w_router)


def _make_sc_top2(tokens, rows_per_worker):
    info = plsc.get_sparse_core_info()
    num_cores = info.num_cores
    mesh = plsc.VectorSubcoreMesh(core_axis_name="c", subcore_axis_name="s")
    num_blocks = rows_per_worker // LANES

    @functools.partial(
        pl.kernel,
        mesh=mesh,
        out_type=[
            jax.ShapeDtypeStruct((tokens * TOP2,), jnp.float32),
            jax.ShapeDtypeStruct((tokens * TOP2,), jnp.int32),
        ],
        scratch_types=[
            pltpu.VMEM((N_EXPERTS, rows_per_worker), jnp.float32),
            pltpu.VMEM((rows_per_worker * TOP2,), jnp.float32),
            pltpu.VMEM((rows_per_worker * TOP2,), jnp.int32),
        ],
    )
    def top2_kernel(pt_hbm, w_hbm, i_hbm, pt_v, wf_v, if_v):
        wid = lax.axis_index("s") * num_cores + lax.axis_index("c")
        base = wid * rows_per_worker
        pltpu.sync_copy(pt_hbm.at[wid], pt_v)

        lanes = lax.iota(jnp.int32, LANES)
        even = lax.rem(lanes, 2) == 0
        half = lax.shift_right_logical(lanes, 1)
        lo_idx = half
        hi_idx = half + LANES // 2
        expert_ids = [jnp.full((LANES,), e, jnp.int32) for e in range(N_EXPERTS)]

        def take(v, idx):
            return lax.gather(
                v, idx[:, None],
                lax.GatherDimensionNumbers(offset_dims=(),
                                           collapsed_slice_dims=(0,),
                                           start_index_map=(0,)),
                (1,),
                mode=lax.GatherScatterMode.PROMISE_IN_BOUNDS)

        def block(b, carry):
            row0 = b * LANES
            m1 = pt_v[0, pl.ds(row0, LANES)]
            i1 = expert_ids[0]
            m2 = jnp.full((LANES,), -1.0, jnp.float32)
            i2 = expert_ids[0]
            for e in range(1, N_EXPERTS):
                pe = pt_v[e, pl.ds(row0, LANES)]
                is1 = pe > m1
                lose_v = jnp.where(is1, m1, pe)
                lose_i = jnp.where(is1, i1, expert_ids[e])
                m1 = jnp.where(is1, pe, m1)
                i1 = jnp.where(is1, expert_ids[e], i1)
                is2 = lose_v > m2
                m2 = jnp.where(is2, lose_v, m2)
                i2 = jnp.where(is2, lose_i, i2)
            inv = 1.0 / (m1 + m2)
            w1 = m1 * inv
            w2 = m2 * inv
            flat0 = row0 * TOP2
            wf_v[pl.ds(flat0, LANES)] = jnp.where(
                even, take(w1, lo_idx), take(w2, lo_idx))
            wf_v[pl.ds(flat0 + LANES, LANES)] = jnp.where(
                even, take(w1, hi_idx), take(w2, hi_idx))
            if_v[pl.ds(flat0, LANES)] = jnp.where(
                even, take(i1, lo_idx), take(i2, lo_idx))
            if_v[pl.ds(flat0 + LANES, LANES)] = jnp.where(
                even, take(i1, hi_idx), take(i2, hi_idx))
            return carry

        lax.fori_loop(0, num_blocks, block, 0)
        out_sl = pl.ds(base * TOP2, rows_per_worker * TOP2)
        pltpu.sync_copy(wf_v, w_hbm.at[out_sl])
        pltpu.sync_copy(if_v, i_hbm.at[out_sl])

    return top2_kernel


def kernel(x, w_router):
    tokens = x.shape[0]
    info = plsc.get_sparse_core_info()
    num_workers = info.num_cores * info.num_subcores
    rows_per_worker = tokens // num_workers
    probs, probs_t = _router_probs(x, w_router, rows_per_worker)
    top2 = _make_sc_top2(tokens, rows_per_worker)
    w_flat, i_flat = top2(probs_t)
    return (w_flat.reshape(tokens, TOP2), i_flat.reshape(tokens, TOP2), probs)
